# baseline (speedup 1.0000x reference)
import functools

import jax
import jax.numpy as jnp
from jax import lax
from jax.experimental import pallas as pl
from jax.experimental.pallas import tpu as pltpu
from jax.experimental.pallas import tpu_sc as plsc

RAW = 2562
NEW = 10242
C = 256
K7 = 7 * C
IN_CH = 512

TOP_PAD = 2688
DOWN = 7680
NPAD = 10752
SHIFT = TOP_PAD - RAW
B3 = 7 * NPAD
NW = 32
CHUNK = 112

M1 = 2688
MBLK = 672
NBLK = NPAD // MBLK


NBUF = 3


def _make_sc_gather(T, B, D=C, dtype=jnp.float32):
    bpw = B // NW
    nch = bpw // CHUNK
    mesh = plsc.VectorSubcoreMesh(core_axis_name="c", subcore_axis_name="s")

    def body(table, idx, out, idx_v, b0, b1, b2, g0, g1, g2, w0, w1, w2):
        bufs = (b0, b1, b2)
        gsems = (g0, g1, g2)
        wsems = (w0, w1, w2)
        cc = lax.axis_index("c")
        ss = lax.axis_index("s")
        wid = ss * 2 + cc
        base0 = pl.multiple_of(wid * bpw, 8)
        pltpu.sync_copy(idx.at[pl.ds(base0, bpw)], idx_v)
        gh = [None] * nch
        wh = [None] * nch
        for k in range(nch):
            b = k % NBUF
            if k >= NBUF:
                wh[k - NBUF].wait()
            gh[k] = pltpu.async_copy(
                table.at[idx_v.at[pl.ds(k * CHUNK, CHUNK)]], bufs[b], gsems[b]
            )
            if k >= 1:
                pb = (k - 1) % NBUF
                gh[k - 1].wait()
                wh[k - 1] = pltpu.async_copy(
                    bufs[pb],
                    out.at[pl.ds(pl.multiple_of(base0 + (k - 1) * CHUNK, 8), CHUNK)],
                    wsems[pb],
                )
        gh[nch - 1].wait()
        lb = (nch - 1) % NBUF
        wh[nch - 1] = pltpu.async_copy(
            bufs[lb],
            out.at[pl.ds(pl.multiple_of(base0 + (nch - 1) * CHUNK, 8), CHUNK)],
            wsems[lb],
        )
        for k in range(max(0, nch - NBUF), nch):
            wh[k].wait()

    return pl.kernel(
        body,
        mesh=mesh,
        out_type=jax.ShapeDtypeStruct((B, D), dtype),
        scratch_types=[
            pltpu.VMEM((bpw,), jnp.int32),
            pltpu.VMEM((CHUNK, D), dtype),
            pltpu.VMEM((CHUNK, D), dtype),
            pltpu.VMEM((CHUNK, D), dtype),
            pltpu.SemaphoreType.DMA,
            pltpu.SemaphoreType.DMA,
            pltpu.SemaphoreType.DMA,
            pltpu.SemaphoreType.DMA,
            pltpu.SemaphoreType.DMA,
            pltpu.SemaphoreType.DMA,
        ],
    )


def _up_mm_body(x_ref, w_ref, b_ref, o_ref):
    o_ref[...] = (
        jnp.dot(x_ref[...], w_ref[...], preferred_element_type=jnp.float32)
        + b_ref[...]
    )


def _assemble_body(ge_ref, go_ref, sl_ref, sr_ref, o_ref):
    i = pl.program_id(0)

    @pl.when(i < TOP_PAD // MBLK)
    def _top():
        o_ref[...] = ge_ref[...].astype(o_ref.dtype)

    @pl.when(i >= TOP_PAD // MBLK)
    def _down():
        o_ref[...] = (
            jnp.dot(ge_ref[...], sl_ref[...], preferred_element_type=jnp.float32)
            + jnp.dot(go_ref[...], sr_ref[...], preferred_element_type=jnp.float32)
        ).astype(o_ref.dtype)


def _conv_mm_body(g_ref, w_ref, b_ref, z_ref, st_ref, acc_ref):
    i = pl.program_id(0)
    z = (
        jnp.dot(g_ref[...], w_ref[...], preferred_element_type=jnp.float32)
        + b_ref[...]
    )
    z_ref[...] = z
    rows = i * MBLK + lax.broadcasted_iota(jnp.int32, (MBLK, 1), 0)
    zm = jnp.where(rows < NEW, z, 0.0)

    @pl.when(i == 0)
    def _init():
        acc_ref[...] = jnp.zeros_like(acc_ref)

    acc_ref[0:1, :] += jnp.sum(zm, axis=0, keepdims=True)
    acc_ref[1:2, :] += jnp.sum(zm * zm, axis=0, keepdims=True)

    @pl.when(i == NBLK - 1)
    def _fin():
        st_ref[...] = acc_ref[...]


def _bn_act_body(z_ref, st_ref, gam_ref, bet_ref, o_ref):
    inv_n = 1.0 / NEW
    mean = st_ref[0:1, :] * inv_n
    var = st_ref[1:2, :] * inv_n - mean * mean
    scale = gam_ref[...] * lax.rsqrt(var + 1e-5)
    shift = bet_ref[...] - mean * scale
    a = z_ref[...] * scale + shift
    o_ref[...] = jnp.where(a >= 0, a, 0.2 * a).astype(o_ref.dtype)


def _up_matmul(x1p, W_up, b_up):
    return pl.pallas_call(
        _up_mm_body,
        grid=(7,),
        in_specs=[
            pl.BlockSpec((M1, IN_CH), lambda j: (0, 0)),
            pl.BlockSpec((IN_CH, C), lambda j: (0, j)),
            pl.BlockSpec((1, C), lambda j: (0, j)),
        ],
        out_specs=pl.BlockSpec((M1, C), lambda j: (0, j)),
        out_shape=jax.ShapeDtypeStruct((M1, K7), jnp.float32),
    )(x1p, W_up, b_up.reshape(1, K7))


def _assemble_x(ge, go, sl, sr):
    return pl.pallas_call(
        _assemble_body,
        grid=(NBLK,),
        in_specs=[
            pl.BlockSpec((MBLK, C), lambda i: (i, 0)),
            pl.BlockSpec((MBLK, C), lambda i: (i, 0)),
            pl.BlockSpec((C, C), lambda i: (0, 0)),
            pl.BlockSpec((C, C), lambda i: (0, 0)),
        ],
        out_specs=pl.BlockSpec((MBLK, C), lambda i: (i, 0)),
        out_shape=jax.ShapeDtypeStruct((NPAD, C), jnp.bfloat16),
    )(ge, go, sl, sr)


def _conv_matmul(g, W, b):
    return pl.pallas_call(
        _conv_mm_body,
        grid=(NBLK,),
        in_specs=[
            pl.BlockSpec((MBLK, K7), lambda i: (i, 0)),
            pl.BlockSpec((K7, C), lambda i: (0, 0)),
            pl.BlockSpec((1, C), lambda i: (0, 0)),
        ],
        out_specs=[
            pl.BlockSpec((MBLK, C), lambda i: (i, 0)),
            pl.BlockSpec((2, C), lambda i: (0, 0)),
        ],
        out_shape=[
            jax.ShapeDtypeStruct((NPAD, C), jnp.float32),
            jax.ShapeDtypeStruct((2, C), jnp.float32),
        ],
        scratch_shapes=[pltpu.VMEM((2, C), jnp.float32)],
    )(g.reshape(NPAD, K7), W, b.reshape(1, C))


def _bn_act(z, st, gamma, beta, dtype=jnp.float32):
    return pl.pallas_call(
        _bn_act_body,
        grid=(NBLK,),
        in_specs=[
            pl.BlockSpec((MBLK, C), lambda i: (i, 0)),
            pl.BlockSpec((2, C), lambda i: (0, 0)),
            pl.BlockSpec((1, C), lambda i: (0, 0)),
            pl.BlockSpec((1, C), lambda i: (0, 0)),
        ],
        out_specs=pl.BlockSpec((MBLK, C), lambda i: (i, 0)),
        out_shape=jax.ShapeDtypeStruct((NPAD, C), dtype),
    )(z, st, gamma.reshape(1, C), beta.reshape(1, C))


def _sc_gather(table, idx, B):
    return _make_sc_gather(
        table.shape[0], B, D=table.shape[1], dtype=table.dtype
    )(table, idx)


def kernel(x1, W_up, b_up, W_c1, b_c1, gamma1, beta1, W_c2, b_c2, gamma2,
           beta2, upconv_top_index, upconv_down_index, neigh_orders):
    i32 = jnp.int32
    top = upconv_top_index.astype(i32)
    dn = upconv_down_index.astype(i32).reshape(-1, 2)
    neigh = neigh_orders.astype(i32)

    zpad_top = jnp.zeros((SHIFT,), i32)
    zpad_dn = jnp.zeros((NPAD - TOP_PAD - DOWN,), i32)
    eidx = jnp.concatenate([top, zpad_top, dn[:, 0], zpad_dn])
    oidx = jnp.concatenate([top, zpad_top, dn[:, 1], zpad_dn])

    neigh1 = jnp.where(neigh >= RAW, neigh + SHIFT, neigh)
    zpad_g = jnp.zeros((B3 - 7 * NEW,), i32)
    nidx1 = jnp.concatenate([neigh1, zpad_g])
    nidx2 = jnp.concatenate([neigh, zpad_g])

    ccol = jnp.arange(C)[:, None] // 2
    krow = jnp.arange(C)[None, :]
    sl = jnp.where(ccol == krow, 0.5, 0.0).astype(jnp.float32)
    sr = jnp.where(ccol == (krow - 128), 0.5, 0.0).astype(jnp.float32)

    x1p = jnp.pad(x1, ((0, M1 - RAW), (0, 0)))

    up_flat = _up_matmul(x1p, W_up, b_up).reshape(M1 * 7, C)

    def to_i32(t):
        return lax.bitcast_convert_type(t.reshape(t.shape[0], C // 2, 2), jnp.int32)

    def to_bf(t):
        return lax.bitcast_convert_type(t, jnp.bfloat16).reshape(t.shape[0], C)

    ge = _sc_gather(up_flat, eidx, NPAD)
    go = _sc_gather(up_flat, oidx, NPAD)
    x = _assemble_x(ge, go, sl, sr)

    g1 = to_bf(_sc_gather(to_i32(x), nidx1, B3))
    z1, st1 = _conv_matmul(g1, W_c1.astype(jnp.bfloat16), b_c1)
    a1 = _bn_act(z1, st1, gamma1, beta1, dtype=jnp.bfloat16)

    g2 = to_bf(_sc_gather(to_i32(a1), nidx2, B3))
    z2, st2 = _conv_matmul(g2, W_c2.astype(jnp.bfloat16), b_c2)
    h2 = _bn_act(z2, st2, gamma2, beta2)

    return h2[:NEW]


# speedup vs baseline: 27.5308x; 27.5308x over previous
import jax
import jax.numpy as jnp
from jax import lax
from jax.experimental import pallas as pl
from jax.experimental.pallas import tpu as pltpu
from jax.experimental.pallas import tpu_sc as plsc

RAW = 2562
NEW = 10242
C = 256
H = 128
K7 = 7 * C
KP = 7 * H
IN_CH = 512

TOP_PAD = 2688
DOWN = 7680
NPAD = 10752
SHIFT = TOP_PAD - RAW
B3 = 7 * NPAD
NW = 32
CHUNK = 112

M1 = 2688
MBLK = 672
NBLK = NPAD // MBLK

_HI = -65536


def _rne16(i):
    return i + 0x7FFF + ((i >> 16) & 1)


def _pack(left, right):
    li = _rne16(lax.bitcast_convert_type(left, jnp.int32))
    ri = _rne16(lax.bitcast_convert_type(right, jnp.int32))
    return ((li >> 16) & 0xFFFF) | (ri & _HI)


def _unpack_lo(x):
    return lax.bitcast_convert_type(lax.shift_left(x, 16), jnp.float32)


def _unpack_hi(x):
    return lax.bitcast_convert_type(lax.bitwise_and(x, jnp.full_like(x, _HI)), jnp.float32)


NBUF = 3


def _make_sc_gather(T, B):
    bpw = B // NW
    nch = bpw // CHUNK
    mesh = plsc.VectorSubcoreMesh(core_axis_name="c", subcore_axis_name="s")

    def body(table, idx, out, idx_v, b0, b1, b2, g0, g1, g2, w0, w1, w2):
        bufs = (b0, b1, b2)
        gsems = (g0, g1, g2)
        wsems = (w0, w1, w2)
        cc = lax.axis_index("c")
        ss = lax.axis_index("s")
        wid = ss * 2 + cc
        base0 = pl.multiple_of(wid * bpw, 8)
        pltpu.sync_copy(idx.at[pl.ds(base0, bpw)], idx_v)
        gh = [None] * nch
        wh = [None] * nch
        for k in range(nch):
            b = k % NBUF
            if k >= NBUF:
                wh[k - NBUF].wait()
            gh[k] = pltpu.async_copy(
                table.at[idx_v.at[pl.ds(k * CHUNK, CHUNK)]], bufs[b], gsems[b]
            )
            if k >= 1:
                pb = (k - 1) % NBUF
                gh[k - 1].wait()
                wh[k - 1] = pltpu.async_copy(
                    bufs[pb],
                    out.at[pl.ds(pl.multiple_of(base0 + (k - 1) * CHUNK, 8), CHUNK)],
                    wsems[pb],
                )
        gh[nch - 1].wait()
        lb = (nch - 1) % NBUF
        wh[nch - 1] = pltpu.async_copy(
            bufs[lb],
            out.at[pl.ds(pl.multiple_of(base0 + (nch - 1) * CHUNK, 8), CHUNK)],
            wsems[lb],
        )
        for k in range(max(0, nch - NBUF), nch):
            wh[k].wait()

    return pl.kernel(
        body,
        mesh=mesh,
        out_type=jax.ShapeDtypeStruct((B, H), jnp.int32),
        scratch_types=[
            pltpu.VMEM((bpw,), jnp.int32),
            pltpu.VMEM((CHUNK, H), jnp.int32),
            pltpu.VMEM((CHUNK, H), jnp.int32),
            pltpu.VMEM((CHUNK, H), jnp.int32),
            pltpu.SemaphoreType.DMA,
            pltpu.SemaphoreType.DMA,
            pltpu.SemaphoreType.DMA,
            pltpu.SemaphoreType.DMA,
            pltpu.SemaphoreType.DMA,
            pltpu.SemaphoreType.DMA,
        ],
    )


def _sc_gather(table, idx, B):
    return _make_sc_gather(table.shape[0], B)(table, idx)


def _up_mm_body(x_ref, w_ref, b_ref, o_ref):
    z = (
        jnp.dot(x_ref[...], w_ref[...], preferred_element_type=jnp.float32)
        + b_ref[...]
    )
    o_ref[...] = _pack(z[:, :H], z[:, H:])


def _assemble_body(ge_ref, go_ref, sl_ref, o_ref):
    i = pl.program_id(0)

    @pl.when(i < TOP_PAD // MBLK)
    def _top():
        o_ref[...] = ge_ref[...]

    @pl.when(i >= TOP_PAD // MBLK)
    def _down():
        ge = ge_ref[...]
        go = go_ref[...]
        e = jnp.concatenate([_unpack_lo(ge), _unpack_hi(ge)], axis=1)
        o = jnp.concatenate([_unpack_lo(go), _unpack_hi(go)], axis=1)
        left = jnp.dot(e, sl_ref[...], preferred_element_type=jnp.float32)
        right = jnp.dot(o, sl_ref[...], preferred_element_type=jnp.float32)
        o_ref[...] = _pack(left, right)


def _conv_mm_body(g_ref, wl_ref, wh_ref, b_ref, z_ref, st_ref, acc_ref):
    i = pl.program_id(0)
    g = g_ref[...]
    z = (
        jnp.dot(_unpack_lo(g), wl_ref[...], preferred_element_type=jnp.float32)
        + jnp.dot(_unpack_hi(g), wh_ref[...], preferred_element_type=jnp.float32)
        + b_ref[...]
    )
    z_ref[...] = z
    rows = i * MBLK + lax.broadcasted_iota(jnp.int32, (MBLK, 1), 0)
    zm = jnp.where(rows < NEW, z, 0.0)

    @pl.when(i == 0)
    def _init():
        acc_ref[...] = jnp.zeros_like(acc_ref)

    acc_ref[0:1, :] += jnp.sum(zm, axis=0, keepdims=True)
    acc_ref[1:2, :] += jnp.sum(zm * zm, axis=0, keepdims=True)

    @pl.when(i == NBLK - 1)
    def _fin():
        st_ref[...] = acc_ref[...]


def _bn_act_body(z_ref, st_ref, gam_ref, bet_ref, o_ref):
    inv_n = 1.0 / NEW
    mean = st_ref[0:1, :] * inv_n
    var = st_ref[1:2, :] * inv_n - mean * mean
    scale = gam_ref[...] * lax.rsqrt(var + 1e-5)
    shift = bet_ref[...] - mean * scale
    a = z_ref[...] * scale + shift
    a = jnp.where(a >= 0, a, 0.2 * a)
    if o_ref.shape[1] == H:
        o_ref[...] = _pack(a[:, :H], a[:, H:])
    else:
        o_ref[...] = a


def _up_matmul(x1p, W_up, b_up):
    return pl.pallas_call(
        _up_mm_body,
        grid=(7,),
        in_specs=[
            pl.BlockSpec((M1, IN_CH), lambda j: (0, 0)),
            pl.BlockSpec((IN_CH, C), lambda j: (0, j)),
            pl.BlockSpec((1, C), lambda j: (0, j)),
        ],
        out_specs=pl.BlockSpec((M1, H), lambda j: (0, j)),
        out_shape=jax.ShapeDtypeStruct((M1, KP), jnp.int32),
    )(x1p, W_up, b_up.reshape(1, K7))


def _assemble_x(ge, go, sl):
    return pl.pallas_call(
        _assemble_body,
        grid=(NBLK,),
        in_specs=[
            pl.BlockSpec((MBLK, H), lambda i: (i, 0)),
            pl.BlockSpec((MBLK, H), lambda i: (i, 0)),
            pl.BlockSpec((C, H), lambda i: (0, 0)),
        ],
        out_specs=pl.BlockSpec((MBLK, H), lambda i: (i, 0)),
        out_shape=jax.ShapeDtypeStruct((NPAD, H), jnp.int32),
    )(ge, go, sl)


def _conv_matmul(g, W_lo, W_hi, b):
    return pl.pallas_call(
        _conv_mm_body,
        grid=(NBLK,),
        in_specs=[
            pl.BlockSpec((MBLK, KP), lambda i: (i, 0)),
            pl.BlockSpec((KP, C), lambda i: (0, 0)),
            pl.BlockSpec((KP, C), lambda i: (0, 0)),
            pl.BlockSpec((1, C), lambda i: (0, 0)),
        ],
        out_specs=[
            pl.BlockSpec((MBLK, C), lambda i: (i, 0)),
            pl.BlockSpec((2, C), lambda i: (0, 0)),
        ],
        out_shape=[
            jax.ShapeDtypeStruct((NPAD, C), jnp.float32),
            jax.ShapeDtypeStruct((2, C), jnp.float32),
        ],
        scratch_shapes=[pltpu.VMEM((2, C), jnp.float32)],
    )(g.reshape(NPAD, KP), W_lo, W_hi, b.reshape(1, C))


def _bn_act(z, st, gamma, beta, packed=False):
    return pl.pallas_call(
        _bn_act_body,
        grid=(NBLK,),
        in_specs=[
            pl.BlockSpec((MBLK, C), lambda i: (i, 0)),
            pl.BlockSpec((2, C), lambda i: (0, 0)),
            pl.BlockSpec((1, C), lambda i: (0, 0)),
            pl.BlockSpec((1, C), lambda i: (0, 0)),
        ],
        out_specs=pl.BlockSpec((MBLK, H if packed else C), lambda i: (i, 0)),
        out_shape=jax.ShapeDtypeStruct(
            (NPAD, H if packed else C), jnp.int32 if packed else jnp.float32
        ),
    )(z, st, gamma.reshape(1, C), beta.reshape(1, C))


def _split_w(W):
    w4 = W.reshape(7, 2, H, C)
    return w4[:, 0].reshape(KP, C), w4[:, 1].reshape(KP, C)


def kernel(x1, W_up, b_up, W_c1, b_c1, gamma1, beta1, W_c2, b_c2, gamma2,
           beta2, upconv_top_index, upconv_down_index, neigh_orders):
    i32 = jnp.int32
    top = upconv_top_index.astype(i32)
    dn = upconv_down_index.astype(i32).reshape(-1, 2)
    neigh = neigh_orders.astype(i32)

    zpad_top = jnp.zeros((SHIFT,), i32)
    zpad_dn = jnp.zeros((NPAD - TOP_PAD - DOWN,), i32)
    eidx = jnp.concatenate([top, zpad_top, dn[:, 0], zpad_dn])
    oidx = jnp.concatenate([top, zpad_top, dn[:, 1], zpad_dn])

    neigh1 = jnp.where(neigh >= RAW, neigh + SHIFT, neigh)
    zpad_g = jnp.zeros((B3 - 7 * NEW,), i32)
    nidx1 = jnp.concatenate([neigh1, zpad_g])
    nidx2 = jnp.concatenate([neigh, zpad_g])

    ccol = jnp.arange(C)[:, None] // 2
    krow = jnp.arange(H)[None, :]
    sl = jnp.where(ccol == krow, 0.5, 0.0).astype(jnp.float32)

    x1p = jnp.pad(x1, ((0, M1 - RAW), (0, 0)))

    up_flat = _up_matmul(x1p, W_up, b_up).reshape(M1 * 7, H)

    ge = _sc_gather(up_flat, eidx, NPAD)
    go = _sc_gather(up_flat, oidx, NPAD)
    x = _assemble_x(ge, go, sl)

    wl1, wh1 = _split_w(W_c1)
    g1 = _sc_gather(x, nidx1, B3)
    z1, st1 = _conv_matmul(g1, wl1, wh1, b_c1)
    a1 = _bn_act(z1, st1, gamma1, beta1, packed=True)

    wl2, wh2 = _split_w(W_c2)
    g2 = _sc_gather(a1, nidx2, B3)
    z2, st2 = _conv_matmul(g2, wl2, wh2, b_c2)
    h2 = _bn_act(z2, st2, gamma2, beta2)

    return h2[:NEW]


# speedup vs baseline: 28.5006x; 1.0352x over previous
import jax
import jax.numpy as jnp
from jax import lax
from jax.experimental import pallas as pl
from jax.experimental.pallas import tpu as pltpu
from jax.experimental.pallas import tpu_sc as plsc

RAW = 2562
NEW = 10242
C = 256
H = 128
K7 = 7 * C
KP = 7 * H
IN_CH = 512

TOP_PAD = 2688
DOWN = 7680
NPAD = 10752
SHIFT = TOP_PAD - RAW
B3 = 7 * NPAD
NW = 32
CHUNK = 112

M1 = 2688
MBLK = 672
NBLK = NPAD // MBLK

_HI = -65536


def _rne16(i):
    return i + 0x7FFF + ((i >> 16) & 1)


def _pack(left, right):
    li = _rne16(lax.bitcast_convert_type(left, jnp.int32))
    ri = _rne16(lax.bitcast_convert_type(right, jnp.int32))
    return ((li >> 16) & 0xFFFF) | (ri & _HI)


def _unpack_lo(x):
    return lax.bitcast_convert_type(lax.shift_left(x, 16), jnp.float32)


def _unpack_hi(x):
    return lax.bitcast_convert_type(lax.bitwise_and(x, jnp.full_like(x, _HI)), jnp.float32)


NBUF = 5
LAG = 3


def _pick_chunk(bpw):
    for c in range(128, 7, -8):
        if bpw % c == 0:
            return c
    raise ValueError(bpw)


def _make_sc_gather(T, B):
    bpw = B // NW
    chunk = _pick_chunk(bpw)
    nch = bpw // chunk
    mesh = plsc.VectorSubcoreMesh(core_axis_name="c", subcore_axis_name="s")

    def body(table, idx, out, idx_v, *bufs_sems):
        bufs = bufs_sems[:NBUF]
        gsems = bufs_sems[NBUF:2 * NBUF]
        wsems = bufs_sems[2 * NBUF:3 * NBUF]
        cc = lax.axis_index("c")
        ss = lax.axis_index("s")
        wid = ss * 2 + cc
        base0 = pl.multiple_of(wid * bpw, 8)
        pltpu.sync_copy(idx.at[pl.ds(base0, bpw)], idx_v)
        gh = [None] * nch
        wh = [None] * nch

        def writeback(j):
            gh[j].wait()
            wh[j] = pltpu.async_copy(
                bufs[j % NBUF],
                out.at[pl.ds(pl.multiple_of(base0 + j * chunk, 8), chunk)],
                wsems[j % NBUF],
            )

        for k in range(nch):
            b = k % NBUF
            if k >= NBUF:
                wh[k - NBUF].wait()
            gh[k] = pltpu.async_copy(
                table.at[idx_v.at[pl.ds(k * chunk, chunk)]], bufs[b], gsems[b]
            )
            if k >= LAG:
                writeback(k - LAG)
        for j in range(max(0, nch - LAG), nch):
            writeback(j)
        for j in range(max(0, nch - NBUF), nch):
            wh[j].wait()

    return pl.kernel(
        body,
        mesh=mesh,
        out_type=jax.ShapeDtypeStruct((B, H), jnp.int32),
        scratch_types=(
            [pltpu.VMEM((bpw,), jnp.int32)]
            + [pltpu.VMEM((chunk, H), jnp.int32)] * NBUF
            + [pltpu.SemaphoreType.DMA] * (2 * NBUF)
        ),
    )


def _sc_gather(table, idx, B):
    return _make_sc_gather(table.shape[0], B)(table, idx)


def _up_mm_body(x_ref, w_ref, b_ref, o_ref):
    z = (
        jnp.dot(x_ref[...], w_ref[...], preferred_element_type=jnp.float32)
        + b_ref[...]
    )
    o_ref[...] = _pack(z[:, :H], z[:, H:])


def _assemble_body(ge_ref, go_ref, sl_ref, o_ref):
    i = pl.program_id(0)

    @pl.when(i < TOP_PAD // MBLK)
    def _top():
        o_ref[...] = ge_ref[...]

    @pl.when(i >= TOP_PAD // MBLK)
    def _down():
        ge = ge_ref[...]
        go = go_ref[...]
        e = jnp.concatenate([_unpack_lo(ge), _unpack_hi(ge)], axis=1)
        o = jnp.concatenate([_unpack_lo(go), _unpack_hi(go)], axis=1)
        left = jnp.dot(e, sl_ref[...], preferred_element_type=jnp.float32)
        right = jnp.dot(o, sl_ref[...], preferred_element_type=jnp.float32)
        o_ref[...] = _pack(left, right)


NPADH = NPAD // 2
B3H = B3 // 2
NBLKH = NPADH // MBLK


def _make_conv_mm_body(row0):
    def _conv_mm_body(g_ref, wl_ref, wh_ref, b_ref, z_ref, st_ref, acc_ref):
        i = pl.program_id(0)
        g = g_ref[...]
        z = (
            jnp.dot(_unpack_lo(g), wl_ref[...], preferred_element_type=jnp.float32)
            + jnp.dot(_unpack_hi(g), wh_ref[...], preferred_element_type=jnp.float32)
            + b_ref[...]
        )
        z_ref[...] = z
        rows = row0 + i * MBLK + lax.broadcasted_iota(jnp.int32, (MBLK, 1), 0)
        zm = jnp.where(rows < NEW, z, 0.0)

        @pl.when(i == 0)
        def _init():
            acc_ref[...] = jnp.zeros_like(acc_ref)

        acc_ref[0:1, :] += jnp.sum(zm, axis=0, keepdims=True)
        acc_ref[1:2, :] += jnp.sum(zm * zm, axis=0, keepdims=True)

        @pl.when(i == NBLKH - 1)
        def _fin():
            st_ref[...] = acc_ref[...]

    return _conv_mm_body


def _bn_act_body(z_ref, sta_ref, stb_ref, gam_ref, bet_ref, o_ref):
    st = sta_ref[...] + stb_ref[...]
    inv_n = 1.0 / NEW
    mean = st[0:1, :] * inv_n
    var = st[1:2, :] * inv_n - mean * mean
    scale = gam_ref[...] * lax.rsqrt(var + 1e-5)
    shift = bet_ref[...] - mean * scale
    a = z_ref[...] * scale + shift
    a = jnp.where(a >= 0, a, 0.2 * a)
    if o_ref.shape[1] == H:
        o_ref[...] = _pack(a[:, :H], a[:, H:])
    else:
        o_ref[...] = a


def _up_matmul(x1p, W_up, b_up):
    return pl.pallas_call(
        _up_mm_body,
        grid=(7,),
        in_specs=[
            pl.BlockSpec((M1, IN_CH), lambda j: (0, 0)),
            pl.BlockSpec((IN_CH, C), lambda j: (0, j)),
            pl.BlockSpec((1, C), lambda j: (0, j)),
        ],
        out_specs=pl.BlockSpec((M1, H), lambda j: (0, j)),
        out_shape=jax.ShapeDtypeStruct((M1, KP), jnp.int32),
    )(x1p, W_up, b_up.reshape(1, K7))


def _assemble_x(ge, go, sl):
    return pl.pallas_call(
        _assemble_body,
        grid=(NBLK,),
        in_specs=[
            pl.BlockSpec((MBLK, H), lambda i: (i, 0)),
            pl.BlockSpec((MBLK, H), lambda i: (i, 0)),
            pl.BlockSpec((C, H), lambda i: (0, 0)),
        ],
        out_specs=pl.BlockSpec((MBLK, H), lambda i: (i, 0)),
        out_shape=jax.ShapeDtypeStruct((NPAD, H), jnp.int32),
    )(ge, go, sl)


def _conv_matmul_half(g, W_lo, W_hi, b, row0):
    return pl.pallas_call(
        _make_conv_mm_body(row0),
        grid=(NBLKH,),
        in_specs=[
            pl.BlockSpec((MBLK, KP), lambda i: (i, 0)),
            pl.BlockSpec((KP, C), lambda i: (0, 0)),
            pl.BlockSpec((KP, C), lambda i: (0, 0)),
            pl.BlockSpec((1, C), lambda i: (0, 0)),
        ],
        out_specs=[
            pl.BlockSpec((MBLK, C), lambda i: (i, 0)),
            pl.BlockSpec((2, C), lambda i: (0, 0)),
        ],
        out_shape=[
            jax.ShapeDtypeStruct((NPADH, C), jnp.float32),
            jax.ShapeDtypeStruct((2, C), jnp.float32),
        ],
        scratch_shapes=[pltpu.VMEM((2, C), jnp.float32)],
    )(g.reshape(NPADH, KP), W_lo, W_hi, b.reshape(1, C))


def _conv(x_table, nidx, W, b):
    wl, wh = _split_w(W)
    ga = _sc_gather(x_table, nidx[:B3H], B3H)
    gb = _sc_gather(x_table, nidx[B3H:], B3H)
    za, sta = _conv_matmul_half(ga, wl, wh, b, 0)
    zb, stb = _conv_matmul_half(gb, wl, wh, b, NPADH)
    return za, zb, sta, stb


def _bn_act(z, sta, stb, gamma, beta, packed=False):
    nb = z.shape[0] // MBLK
    return pl.pallas_call(
        _bn_act_body,
        grid=(nb,),
        in_specs=[
            pl.BlockSpec((MBLK, C), lambda i: (i, 0)),
            pl.BlockSpec((2, C), lambda i: (0, 0)),
            pl.BlockSpec((2, C), lambda i: (0, 0)),
            pl.BlockSpec((1, C), lambda i: (0, 0)),
            pl.BlockSpec((1, C), lambda i: (0, 0)),
        ],
        out_specs=pl.BlockSpec((MBLK, H if packed else C), lambda i: (i, 0)),
        out_shape=jax.ShapeDtypeStruct(
            (z.shape[0], H if packed else C), jnp.int32 if packed else jnp.float32
        ),
    )(z, sta, stb, gamma.reshape(1, C), beta.reshape(1, C))


def _split_w(W):
    w4 = W.reshape(7, 2, H, C)
    return w4[:, 0].reshape(KP, C), w4[:, 1].reshape(KP, C)


def kernel(x1, W_up, b_up, W_c1, b_c1, gamma1, beta1, W_c2, b_c2, gamma2,
           beta2, upconv_top_index, upconv_down_index, neigh_orders):
    i32 = jnp.int32
    top = upconv_top_index.astype(i32)
    dn = upconv_down_index.astype(i32).reshape(-1, 2)
    neigh = neigh_orders.astype(i32)

    zpad_top = jnp.zeros((SHIFT,), i32)
    zpad_dn = jnp.zeros((NPAD - TOP_PAD - DOWN,), i32)
    eidx = jnp.concatenate([top, zpad_top, dn[:, 0], zpad_dn])
    oidx = jnp.concatenate([top, zpad_top, dn[:, 1], zpad_dn])

    neigh1 = jnp.where(neigh >= RAW, neigh + SHIFT, neigh)
    zpad_g = jnp.zeros((B3 - 7 * NEW,), i32)
    nidx1 = jnp.concatenate([neigh1, zpad_g])
    nidx2 = jnp.concatenate([neigh, zpad_g])

    ccol = jnp.arange(C)[:, None] // 2
    krow = jnp.arange(H)[None, :]
    sl = jnp.where(ccol == krow, 0.5, 0.0).astype(jnp.float32)

    x1p = jnp.pad(x1, ((0, M1 - RAW), (0, 0)))

    up_flat = _up_matmul(x1p, W_up, b_up).reshape(M1 * 7, H)

    ge = _sc_gather(up_flat, eidx, NPAD)
    go = _sc_gather(up_flat, oidx, NPAD)
    x = _assemble_x(ge, go, sl)

    z1a, z1b, st1a, st1b = _conv(x, nidx1, W_c1, b_c1)
    a1a = _bn_act(z1a, st1a, st1b, gamma1, beta1, packed=True)
    a1b = _bn_act(z1b, st1a, st1b, gamma1, beta1, packed=True)
    a1 = jnp.concatenate([a1a, a1b], axis=0)

    z2a, z2b, st2a, st2b = _conv(a1, nidx2, W_c2, b_c2)
    h2a = _bn_act(z2a, st2a, st2b, gamma2, beta2)
    h2b = _bn_act(z2b, st2a, st2b, gamma2, beta2)

    return jnp.concatenate([h2a, h2b], axis=0)[:NEW]


# speedup vs baseline: 74.0759x; 2.5991x over previous
import jax
import jax.numpy as jnp
from jax import lax
from jax.experimental import pallas as pl
from jax.experimental.pallas import tpu as pltpu
from jax.experimental.pallas import tpu_sc as plsc

RAW = 2562
NEW = 10242
C = 256
H = 128
K7 = 7 * C
KP = 7 * H
IN_CH = 512

TOP_PAD = 2688
DOWN = 7680
NPAD = 10752
SHIFT = TOP_PAD - RAW
B3 = 7 * NPAD
NW = 32
CHUNK = 112

M1 = 2688
MBLK = 672
NBLK = NPAD // MBLK

_HI = -65536


def _rne16(i):
    return i + 0x7FFF + ((i >> 16) & 1)


def _pack(left, right):
    li = _rne16(lax.bitcast_convert_type(left, jnp.int32))
    ri = _rne16(lax.bitcast_convert_type(right, jnp.int32))
    return ((li >> 16) & 0xFFFF) | (ri & _HI)


def _unpack_lo(x):
    return lax.bitcast_convert_type(lax.shift_left(x, 16), jnp.float32)


def _unpack_hi(x):
    return lax.bitcast_convert_type(lax.bitwise_and(x, jnp.full_like(x, _HI)), jnp.float32)


NBUF = 5
LAG = 3


def _pick_chunk(bpw):
    for c in range(128, 7, -8):
        if bpw % c == 0:
            return c
    raise ValueError(bpw)


def _make_sc_gather(T, B):
    bpw = B // NW
    chunk = _pick_chunk(bpw)
    nch = bpw // chunk
    mesh = plsc.VectorSubcoreMesh(core_axis_name="c", subcore_axis_name="s")

    def body(table, idx, out, idx_v, *bufs_sems):
        bufs = bufs_sems[:NBUF]
        gsems = bufs_sems[NBUF:2 * NBUF]
        wsems = bufs_sems[2 * NBUF:3 * NBUF]
        cc = lax.axis_index("c")
        ss = lax.axis_index("s")
        wid = ss * 2 + cc
        base0 = pl.multiple_of(wid * bpw, 8)
        pltpu.sync_copy(idx.at[pl.ds(base0, bpw)], idx_v)
        gh = [None] * nch
        wh = [None] * nch

        def writeback(j):
            gh[j].wait()
            wh[j] = pltpu.async_copy(
                bufs[j % NBUF],
                out.at[pl.ds(pl.multiple_of(base0 + j * chunk, 8), chunk)],
                wsems[j % NBUF],
            )

        for k in range(nch):
            b = k % NBUF
            if k >= NBUF:
                wh[k - NBUF].wait()
            gh[k] = pltpu.async_copy(
                table.at[idx_v.at[pl.ds(k * chunk, chunk)]], bufs[b], gsems[b]
            )
            if k >= LAG:
                writeback(k - LAG)
        for j in range(max(0, nch - LAG), nch):
            writeback(j)
        for j in range(max(0, nch - NBUF), nch):
            wh[j].wait()

    return pl.kernel(
        body,
        mesh=mesh,
        out_type=jax.ShapeDtypeStruct((B, H), jnp.int32),
        scratch_types=(
            [pltpu.VMEM((bpw,), jnp.int32)]
            + [pltpu.VMEM((chunk, H), jnp.int32)] * NBUF
            + [pltpu.SemaphoreType.DMA] * (2 * NBUF)
        ),
    )


def _sc_gather(table, idx, B):
    return _make_sc_gather(table.shape[0], B)(table, idx)


def _up_mm_body(x_ref, w_ref, b_ref, o_ref):
    z = (
        jnp.dot(x_ref[...], w_ref[...], preferred_element_type=jnp.float32)
        + b_ref[...]
    )
    o_ref[...] = _pack(z[:, :H], z[:, H:])


def _assemble_body(ge_ref, go_ref, sl_ref, o_ref):
    i = pl.program_id(0)

    @pl.when(i < TOP_PAD // MBLK)
    def _top():
        o_ref[...] = ge_ref[...]

    @pl.when(i >= TOP_PAD // MBLK)
    def _down():
        ge = ge_ref[...]
        go = go_ref[...]
        e = jnp.concatenate([_unpack_lo(ge), _unpack_hi(ge)], axis=1)
        o = jnp.concatenate([_unpack_lo(go), _unpack_hi(go)], axis=1)
        left = jnp.dot(e, sl_ref[...], preferred_element_type=jnp.float32)
        right = jnp.dot(o, sl_ref[...], preferred_element_type=jnp.float32)
        o_ref[...] = _pack(left, right)


NPADH = NPAD // 2
B3H = B3 // 2
NBLKH = NPADH // MBLK


def _make_conv_mm_body(row0):
    def _conv_mm_body(g0, g1, g2, g3, g4, g5, g6, wl_ref, wh_ref, b_ref,
                      z_ref, st_ref, acc_ref):
        i = pl.program_id(0)
        grefs = (g0, g1, g2, g3, g4, g5, g6)
        z = b_ref[...]
        for k in range(7):
            g = grefs[k][...]
            wl = wl_ref[k * H:(k + 1) * H, :]
            wh = wh_ref[k * H:(k + 1) * H, :]
            z = z + jnp.dot(_unpack_lo(g), wl, preferred_element_type=jnp.float32)
            z = z + jnp.dot(_unpack_hi(g), wh, preferred_element_type=jnp.float32)
        z_ref[...] = z
        rows = row0 + i * MBLK + lax.broadcasted_iota(jnp.int32, (MBLK, 1), 0)
        zm = jnp.where(rows < NEW, z, 0.0)

        @pl.when(i == 0)
        def _init():
            acc_ref[...] = jnp.zeros_like(acc_ref)

        acc_ref[0:1, :] += jnp.sum(zm, axis=0, keepdims=True)
        acc_ref[1:2, :] += jnp.sum(zm * zm, axis=0, keepdims=True)

        @pl.when(i == NBLKH - 1)
        def _fin():
            st_ref[...] = acc_ref[...]

    return _conv_mm_body


def _bn_act_body(z_ref, sta_ref, stb_ref, gam_ref, bet_ref, o_ref):
    st = sta_ref[...] + stb_ref[...]
    inv_n = 1.0 / NEW
    mean = st[0:1, :] * inv_n
    var = st[1:2, :] * inv_n - mean * mean
    scale = gam_ref[...] * lax.rsqrt(var + 1e-5)
    shift = bet_ref[...] - mean * scale
    a = z_ref[...] * scale + shift
    a = jnp.where(a >= 0, a, 0.2 * a)
    if o_ref.shape[1] == H:
        o_ref[...] = _pack(a[:, :H], a[:, H:])
    else:
        o_ref[...] = a


def _up_matmul(x1p, W_up, b_up):
    return pl.pallas_call(
        _up_mm_body,
        grid=(7,),
        in_specs=[
            pl.BlockSpec((M1, IN_CH), lambda j: (0, 0)),
            pl.BlockSpec((IN_CH, C), lambda j: (0, j)),
            pl.BlockSpec((1, C), lambda j: (0, j)),
        ],
        out_specs=pl.BlockSpec((M1, H), lambda j: (j, 0)),
        out_shape=jax.ShapeDtypeStruct((7 * M1, H), jnp.int32),
    )(x1p, W_up, b_up.reshape(1, K7))


def _assemble_x(ge, go, sl):
    return pl.pallas_call(
        _assemble_body,
        grid=(NBLK,),
        in_specs=[
            pl.BlockSpec((MBLK, H), lambda i: (i, 0)),
            pl.BlockSpec((MBLK, H), lambda i: (i, 0)),
            pl.BlockSpec((C, H), lambda i: (0, 0)),
        ],
        out_specs=pl.BlockSpec((MBLK, H), lambda i: (i, 0)),
        out_shape=jax.ShapeDtypeStruct((NPAD, H), jnp.int32),
    )(ge, go, sl)


def _conv_matmul_half(g, W_lo, W_hi, b, row0):
    gspecs = [
        pl.BlockSpec((MBLK, H), (lambda i, kk=k: (kk * NBLKH + i, 0)))
        for k in range(7)
    ]
    return pl.pallas_call(
        _make_conv_mm_body(row0),
        grid=(NBLKH,),
        in_specs=gspecs + [
            pl.BlockSpec((KP, C), lambda i: (0, 0)),
            pl.BlockSpec((KP, C), lambda i: (0, 0)),
            pl.BlockSpec((1, C), lambda i: (0, 0)),
        ],
        out_specs=[
            pl.BlockSpec((MBLK, C), lambda i: (i, 0)),
            pl.BlockSpec((2, C), lambda i: (0, 0)),
        ],
        out_shape=[
            jax.ShapeDtypeStruct((NPADH, C), jnp.float32),
            jax.ShapeDtypeStruct((2, C), jnp.float32),
        ],
        scratch_shapes=[pltpu.VMEM((2, C), jnp.float32)],
    )(*([g] * 7), W_lo, W_hi, b.reshape(1, C))


def _conv(x_table, nidxa, nidxb, W, b):
    wl, wh = _split_w(W)
    ga = _sc_gather(x_table, nidxa, B3H)
    gb = _sc_gather(x_table, nidxb, B3H)
    za, sta = _conv_matmul_half(ga, wl, wh, b, 0)
    zb, stb = _conv_matmul_half(gb, wl, wh, b, NPADH)
    return za, zb, sta, stb


def _bn_act(z, sta, stb, gamma, beta, packed=False):
    nb = z.shape[0] // MBLK
    return pl.pallas_call(
        _bn_act_body,
        grid=(nb,),
        in_specs=[
            pl.BlockSpec((MBLK, C), lambda i: (i, 0)),
            pl.BlockSpec((2, C), lambda i: (0, 0)),
            pl.BlockSpec((2, C), lambda i: (0, 0)),
            pl.BlockSpec((1, C), lambda i: (0, 0)),
            pl.BlockSpec((1, C), lambda i: (0, 0)),
        ],
        out_specs=pl.BlockSpec((MBLK, H if packed else C), lambda i: (i, 0)),
        out_shape=jax.ShapeDtypeStruct(
            (z.shape[0], H if packed else C), jnp.int32 if packed else jnp.float32
        ),
    )(z, sta, stb, gamma.reshape(1, C), beta.reshape(1, C))


def _split_w(W):
    w4 = W.reshape(7, 2, H, C)
    return w4[:, 0].reshape(KP, C), w4[:, 1].reshape(KP, C)


def kernel(x1, W_up, b_up, W_c1, b_c1, gamma1, beta1, W_c2, b_c2, gamma2,
           beta2, upconv_top_index, upconv_down_index, neigh_orders):
    i32 = jnp.int32
    top = upconv_top_index.astype(i32)
    dn = upconv_down_index.astype(i32).reshape(-1, 2)
    neigh = neigh_orders.astype(i32)

    def kmaj_up(r):
        return (r % 7) * M1 + r // 7

    zpad_top = jnp.arange(SHIFT, dtype=i32)
    zpad_dn = jnp.arange(NPAD - TOP_PAD - DOWN, dtype=i32)
    eidx = jnp.concatenate([kmaj_up(top), zpad_top, kmaj_up(dn[:, 0]), zpad_dn])
    oidx = jnp.concatenate([kmaj_up(top), zpad_top, kmaj_up(dn[:, 1]), zpad_dn])

    neigh1 = jnp.where(neigh >= RAW, neigh + SHIFT, neigh)
    padrows = (jnp.arange((NPAD - NEW) * 7, dtype=i32) % NPAD).reshape(-1, 7)

    def kmaj_conv(nn):
        full = jnp.concatenate([nn.reshape(NEW, 7), padrows], axis=0)
        a = full[:NPADH].T.reshape(B3H)
        b = full[NPADH:].T.reshape(B3H)
        return a, b

    n1a, n1b = kmaj_conv(neigh1)
    n2a, n2b = kmaj_conv(neigh)

    ccol = jnp.arange(C)[:, None] // 2
    krow = jnp.arange(H)[None, :]
    sl = jnp.where(ccol == krow, 0.5, 0.0).astype(jnp.float32)

    x1p = jnp.pad(x1, ((0, M1 - RAW), (0, 0)))

    up_flat = _up_matmul(x1p, W_up, b_up)

    ge = _sc_gather(up_flat, eidx, NPAD)
    go = _sc_gather(up_flat, oidx, NPAD)
    x = _assemble_x(ge, go, sl)

    z1a, z1b, st1a, st1b = _conv(x, n1a, n1b, W_c1, b_c1)
    a1a = _bn_act(z1a, st1a, st1b, gamma1, beta1, packed=True)
    a1b = _bn_act(z1b, st1a, st1b, gamma1, beta1, packed=True)
    a1 = jnp.concatenate([a1a, a1b], axis=0)

    z2a, z2b, st2a, st2b = _conv(a1, n2a, n2b, W_c2, b_c2)
    h2a = _bn_act(z2a, st2a, st2b, gamma2, beta2)
    h2b = _bn_act(z2b, st2a, st2b, gamma2, beta2)

    return jnp.concatenate([h2a, h2b], axis=0)[:NEW]


# speedup vs baseline: 79.1498x; 1.0685x over previous
import jax
import jax.numpy as jnp
from jax import lax
from jax.experimental import pallas as pl
from jax.experimental.pallas import tpu as pltpu
from jax.experimental.pallas import tpu_sc as plsc

RAW = 2562
NEW = 10242
C = 256
H = 128
K7 = 7 * C
KP = 7 * H
IN_CH = 512

TOP_PAD = 2688
DOWN = 7680
NPAD = 10752
SHIFT = TOP_PAD - RAW
B3 = 7 * NPAD
NW = 32
CHUNK = 112

M1 = 2688
MBLK = 672
NBLK = NPAD // MBLK

_HI = -65536


def _rne16(i):
    return i + 0x7FFF + ((i >> 16) & 1)


def _pack(left, right):
    li = _rne16(lax.bitcast_convert_type(left, jnp.int32))
    ri = _rne16(lax.bitcast_convert_type(right, jnp.int32))
    return ((li >> 16) & 0xFFFF) | (ri & _HI)


def _unpack_lo(x):
    return lax.bitcast_convert_type(lax.shift_left(x, 16), jnp.float32)


def _unpack_hi(x):
    return lax.bitcast_convert_type(lax.bitwise_and(x, jnp.full_like(x, _HI)), jnp.float32)


NBUF = 5
LAG = 3


def _pick_chunk(bpw):
    for c in range(128, 7, -8):
        if bpw % c == 0:
            return c
    raise ValueError(bpw)


def _make_sc_gather(T, B):
    bpw = B // NW
    chunk = _pick_chunk(bpw)
    nch = bpw // chunk
    mesh = plsc.VectorSubcoreMesh(core_axis_name="c", subcore_axis_name="s")

    def body(table, idx, out, idx_v, *bufs_sems):
        bufs = bufs_sems[:NBUF]
        gsems = bufs_sems[NBUF:2 * NBUF]
        wsems = bufs_sems[2 * NBUF:3 * NBUF]
        cc = lax.axis_index("c")
        ss = lax.axis_index("s")
        wid = ss * 2 + cc
        base0 = pl.multiple_of(wid * bpw, 8)
        pltpu.sync_copy(idx.at[pl.ds(base0, bpw)], idx_v)
        gh = [None] * nch
        wh = [None] * nch

        def writeback(j):
            gh[j].wait()
            wh[j] = pltpu.async_copy(
                bufs[j % NBUF],
                out.at[pl.ds(pl.multiple_of(base0 + j * chunk, 8), chunk)],
                wsems[j % NBUF],
            )

        for k in range(nch):
            b = k % NBUF
            if k >= NBUF:
                wh[k - NBUF].wait()
            gh[k] = pltpu.async_copy(
                table.at[idx_v.at[pl.ds(k * chunk, chunk)]], bufs[b], gsems[b]
            )
            if k >= LAG:
                writeback(k - LAG)
        for j in range(max(0, nch - LAG), nch):
            writeback(j)
        for j in range(max(0, nch - NBUF), nch):
            wh[j].wait()

    return pl.kernel(
        body,
        mesh=mesh,
        out_type=jax.ShapeDtypeStruct((B, H), jnp.int32),
        scratch_types=(
            [pltpu.VMEM((bpw,), jnp.int32)]
            + [pltpu.VMEM((chunk, H), jnp.int32)] * NBUF
            + [pltpu.SemaphoreType.DMA] * (2 * NBUF)
        ),
    )


def _sc_gather(table, idx, B):
    return _make_sc_gather(table.shape[0], B)(table, idx)


def _up_mm_body(x_ref, w_ref, b_ref, o_ref):
    z = (
        jnp.dot(x_ref[...], w_ref[...], preferred_element_type=jnp.float32)
        + b_ref[...]
    )
    o_ref[...] = _pack(z[:, :H], z[:, H:])


def _assemble_body(ge_ref, go_ref, sl_ref, o_ref):
    i = pl.program_id(0)

    @pl.when(i < TOP_PAD // MBLK)
    def _top():
        o_ref[...] = ge_ref[...]

    @pl.when(i >= TOP_PAD // MBLK)
    def _down():
        ge = ge_ref[...]
        go = go_ref[...]
        e = jnp.concatenate([_unpack_lo(ge), _unpack_hi(ge)], axis=1)
        o = jnp.concatenate([_unpack_lo(go), _unpack_hi(go)], axis=1)
        left = jnp.dot(e, sl_ref[...], preferred_element_type=jnp.float32)
        right = jnp.dot(o, sl_ref[...], preferred_element_type=jnp.float32)
        o_ref[...] = _pack(left, right)


NPADH = NPAD // 2
B3H = B3 // 2
NBLKH = NPADH // MBLK


def _make_conv_mm_body(row0):
    def _conv_mm_body(g0, g1, g2, g3, g4, g5, g6, wl_ref, wh_ref, b_ref,
                      z_ref, st_ref, acc_ref):
        i = pl.program_id(0)
        grefs = (g0, g1, g2, g3, g4, g5, g6)
        z = b_ref[...]
        for k in range(7):
            g = grefs[k][...]
            wl = wl_ref[k * H:(k + 1) * H, :]
            wh = wh_ref[k * H:(k + 1) * H, :]
            z = z + jnp.dot(_unpack_lo(g), wl, preferred_element_type=jnp.float32)
            z = z + jnp.dot(_unpack_hi(g), wh, preferred_element_type=jnp.float32)
        z_ref[...] = _pack(z[:, :H], z[:, H:])
        rows = row0 + i * MBLK + lax.broadcasted_iota(jnp.int32, (MBLK, 1), 0)
        zm = jnp.where(rows < NEW, z, 0.0)

        @pl.when(i == 0)
        def _init():
            acc_ref[...] = jnp.zeros_like(acc_ref)

        acc_ref[0:1, :] += jnp.sum(zm, axis=0, keepdims=True)
        acc_ref[1:2, :] += jnp.sum(zm * zm, axis=0, keepdims=True)

        @pl.when(i == NBLKH - 1)
        def _fin():
            st_ref[...] = acc_ref[...]

    return _conv_mm_body


def _bn_act_body(za_ref, zb_ref, sta_ref, stb_ref, gam_ref, bet_ref, o_ref):
    i = pl.program_id(0)
    zp = jnp.where(i < NBLKH, za_ref[...], zb_ref[...])
    z = jnp.concatenate([_unpack_lo(zp), _unpack_hi(zp)], axis=1)
    st = sta_ref[...] + stb_ref[...]
    inv_n = 1.0 / NEW
    mean = st[0:1, :] * inv_n
    var = st[1:2, :] * inv_n - mean * mean
    scale = gam_ref[...] * lax.rsqrt(var + 1e-5)
    shift = bet_ref[...] - mean * scale
    a = z * scale + shift
    a = jnp.where(a >= 0, a, 0.2 * a)
    if o_ref.shape[1] == H:
        o_ref[...] = _pack(a[:, :H], a[:, H:])
    else:
        o_ref[...] = a


def _up_matmul(x1p, W_up, b_up):
    return pl.pallas_call(
        _up_mm_body,
        grid=(7,),
        in_specs=[
            pl.BlockSpec((M1, IN_CH), lambda j: (0, 0)),
            pl.BlockSpec((IN_CH, C), lambda j: (0, j)),
            pl.BlockSpec((1, C), lambda j: (0, j)),
        ],
        out_specs=pl.BlockSpec((M1, H), lambda j: (j, 0)),
        out_shape=jax.ShapeDtypeStruct((7 * M1, H), jnp.int32),
    )(x1p, W_up, b_up.reshape(1, K7))


def _assemble_x(ge, go, sl):
    return pl.pallas_call(
        _assemble_body,
        grid=(NBLK,),
        in_specs=[
            pl.BlockSpec((MBLK, H), lambda i: (i, 0)),
            pl.BlockSpec((MBLK, H), lambda i: (i, 0)),
            pl.BlockSpec((C, H), lambda i: (0, 0)),
        ],
        out_specs=pl.BlockSpec((MBLK, H), lambda i: (i, 0)),
        out_shape=jax.ShapeDtypeStruct((NPAD, H), jnp.int32),
    )(ge, go, sl)


def _conv_matmul_half(g, W_lo, W_hi, b, row0):
    gspecs = [
        pl.BlockSpec((MBLK, H), (lambda i, kk=k: (kk * NBLKH + i, 0)))
        for k in range(7)
    ]
    return pl.pallas_call(
        _make_conv_mm_body(row0),
        grid=(NBLKH,),
        in_specs=gspecs + [
            pl.BlockSpec((KP, C), lambda i: (0, 0)),
            pl.BlockSpec((KP, C), lambda i: (0, 0)),
            pl.BlockSpec((1, C), lambda i: (0, 0)),
        ],
        out_specs=[
            pl.BlockSpec((MBLK, H), lambda i: (i, 0)),
            pl.BlockSpec((2, C), lambda i: (0, 0)),
        ],
        out_shape=[
            jax.ShapeDtypeStruct((NPADH, H), jnp.int32),
            jax.ShapeDtypeStruct((2, C), jnp.float32),
        ],
        scratch_shapes=[pltpu.VMEM((2, C), jnp.float32)],
    )(*([g] * 7), W_lo, W_hi, b.reshape(1, C))


def _conv(x_table, nidxa, nidxb, W, b):
    wl, wh = _split_w(W)
    ga = _sc_gather(x_table, nidxa, B3H)
    gb = _sc_gather(x_table, nidxb, B3H)
    za, sta = _conv_matmul_half(ga, wl, wh, b, 0)
    zb, stb = _conv_matmul_half(gb, wl, wh, b, NPADH)
    return za, zb, sta, stb


def _bn_act(za, zb, sta, stb, gamma, beta, packed, out_rows):
    nb = (out_rows + MBLK - 1) // MBLK
    return pl.pallas_call(
        _bn_act_body,
        grid=(nb,),
        in_specs=[
            pl.BlockSpec((MBLK, H), lambda i: (jnp.minimum(i, NBLKH - 1), 0)),
            pl.BlockSpec((MBLK, H), lambda i: (jnp.maximum(i - NBLKH, 0), 0)),
            pl.BlockSpec((2, C), lambda i: (0, 0)),
            pl.BlockSpec((2, C), lambda i: (0, 0)),
            pl.BlockSpec((1, C), lambda i: (0, 0)),
            pl.BlockSpec((1, C), lambda i: (0, 0)),
        ],
        out_specs=pl.BlockSpec((MBLK, H if packed else C), lambda i: (i, 0)),
        out_shape=jax.ShapeDtypeStruct(
            (out_rows, H if packed else C), jnp.int32 if packed else jnp.float32
        ),
    )(za, zb, sta, stb, gamma.reshape(1, C), beta.reshape(1, C))


def _split_w(W):
    w4 = W.reshape(7, 2, H, C)
    return w4[:, 0].reshape(KP, C), w4[:, 1].reshape(KP, C)


def kernel(x1, W_up, b_up, W_c1, b_c1, gamma1, beta1, W_c2, b_c2, gamma2,
           beta2, upconv_top_index, upconv_down_index, neigh_orders):
    i32 = jnp.int32
    top = upconv_top_index.astype(i32)
    dn = upconv_down_index.astype(i32).reshape(-1, 2)
    neigh = neigh_orders.astype(i32)

    def kmaj_up(r):
        return (r % 7) * M1 + r // 7

    zpad_top = jnp.arange(SHIFT, dtype=i32)
    zpad_dn = jnp.arange(NPAD - TOP_PAD - DOWN, dtype=i32)
    eidx = jnp.concatenate([kmaj_up(top), zpad_top, kmaj_up(dn[:, 0]), zpad_dn])
    oidx = jnp.concatenate([kmaj_up(top), zpad_top, kmaj_up(dn[:, 1]), zpad_dn])

    neigh1 = jnp.where(neigh >= RAW, neigh + SHIFT, neigh)
    padrows = (jnp.arange((NPAD - NEW) * 7, dtype=i32) % NPAD).reshape(-1, 7)

    def kmaj_conv(nn):
        full = jnp.concatenate([nn.reshape(NEW, 7), padrows], axis=0)
        a = full[:NPADH].T.reshape(B3H)
        b = full[NPADH:].T.reshape(B3H)
        return a, b

    n1a, n1b = kmaj_conv(neigh1)
    n2a, n2b = kmaj_conv(neigh)

    ccol = jnp.arange(C)[:, None] // 2
    krow = jnp.arange(H)[None, :]
    sl = jnp.where(ccol == krow, 0.5, 0.0).astype(jnp.float32)

    x1p = jnp.pad(x1, ((0, M1 - RAW), (0, 0)))

    up_flat = _up_matmul(x1p, W_up, b_up)

    ge = _sc_gather(up_flat, eidx, NPAD)
    go = _sc_gather(up_flat, oidx, NPAD)
    x = _assemble_x(ge, go, sl)

    z1a, z1b, st1a, st1b = _conv(x, n1a, n1b, W_c1, b_c1)
    a1 = _bn_act(z1a, z1b, st1a, st1b, gamma1, beta1, True, NPAD)

    z2a, z2b, st2a, st2b = _conv(a1, n2a, n2b, W_c2, b_c2)
    return _bn_act(z2a, z2b, st2a, st2b, gamma2, beta2, False, NEW)


# speedup vs baseline: 84.3876x; 1.0662x over previous
import jax
import jax.numpy as jnp
from jax import lax
from jax.experimental import pallas as pl
from jax.experimental.pallas import tpu as pltpu
from jax.experimental.pallas import tpu_sc as plsc

RAW = 2562
NEW = 10242
C = 256
H = 128
K7 = 7 * C
KP = 7 * H
IN_CH = 512

TOP_PAD = 2688
DOWN = 7680
NPAD = 10752
SHIFT = TOP_PAD - RAW
B3 = 7 * NPAD
NW = 32
CHUNK = 112

M1 = 2688
MBLK = 672
NBLK = NPAD // MBLK

_HI = -65536


def _rne16(i):
    return i + 0x7FFF + ((i >> 16) & 1)


def _pack(left, right):
    li = _rne16(lax.bitcast_convert_type(left, jnp.int32))
    ri = _rne16(lax.bitcast_convert_type(right, jnp.int32))
    return ((li >> 16) & 0xFFFF) | (ri & _HI)


def _unpack_lo(x):
    return lax.bitcast_convert_type(lax.shift_left(x, 16), jnp.float32)


def _unpack_hi(x):
    return lax.bitcast_convert_type(lax.bitwise_and(x, jnp.full_like(x, _HI)), jnp.float32)


NBUF = 5
LAG = 3


def _pick_chunk(bpw):
    for c in range(128, 7, -8):
        if bpw % c == 0:
            return c
    raise ValueError(bpw)


def _make_sc_gather(T, B):
    bpw = B // NW
    chunk = _pick_chunk(bpw)
    nch = bpw // chunk
    mesh = plsc.VectorSubcoreMesh(core_axis_name="c", subcore_axis_name="s")

    def body(table, idx, out, idx_v, *bufs_sems):
        bufs = bufs_sems[:NBUF]
        gsems = bufs_sems[NBUF:2 * NBUF]
        wsems = bufs_sems[2 * NBUF:3 * NBUF]
        cc = lax.axis_index("c")
        ss = lax.axis_index("s")
        wid = ss * 2 + cc
        base0 = pl.multiple_of(wid * bpw, 8)
        pltpu.sync_copy(idx.at[pl.ds(base0, bpw)], idx_v)
        gh = [None] * nch
        wh = [None] * nch

        def writeback(j):
            gh[j].wait()
            wh[j] = pltpu.async_copy(
                bufs[j % NBUF],
                out.at[pl.ds(pl.multiple_of(base0 + j * chunk, 8), chunk)],
                wsems[j % NBUF],
            )

        for k in range(nch):
            b = k % NBUF
            if k >= NBUF:
                wh[k - NBUF].wait()
            gh[k] = pltpu.async_copy(
                table.at[idx_v.at[pl.ds(k * chunk, chunk)]], bufs[b], gsems[b]
            )
            if k >= LAG:
                writeback(k - LAG)
        for j in range(max(0, nch - LAG), nch):
            writeback(j)
        for j in range(max(0, nch - NBUF), nch):
            wh[j].wait()

    return pl.kernel(
        body,
        mesh=mesh,
        out_type=jax.ShapeDtypeStruct((B, H), jnp.int32),
        scratch_types=(
            [pltpu.VMEM((bpw,), jnp.int32)]
            + [pltpu.VMEM((chunk, H), jnp.int32)] * NBUF
            + [pltpu.SemaphoreType.DMA] * (2 * NBUF)
        ),
    )


def _make_sc_gather2(T, B):
    bpw = B // NW
    chunk = _pick_chunk(bpw)
    nch = bpw // chunk
    mesh = plsc.VectorSubcoreMesh(core_axis_name="c", subcore_axis_name="s")

    def body(table, idxe, idxo, oute, outo, idx_v, *bufs_sems):
        bufs = bufs_sems[:NBUF]
        gsems = bufs_sems[NBUF:2 * NBUF]
        wsems = bufs_sems[2 * NBUF:3 * NBUF]
        cc = lax.axis_index("c")
        ss = lax.axis_index("s")
        wid = ss * 2 + cc
        base0 = pl.multiple_of(wid * bpw, 8)
        for idx, out in ((idxe, oute), (idxo, outo)):
            pltpu.sync_copy(idx.at[pl.ds(base0, bpw)], idx_v)
            gh = [None] * nch
            wh = [None] * nch

            def writeback(j):
                gh[j].wait()
                wh[j] = pltpu.async_copy(
                    bufs[j % NBUF],
                    out.at[pl.ds(pl.multiple_of(base0 + j * chunk, 8), chunk)],
                    wsems[j % NBUF],
                )

            for k in range(nch):
                b = k % NBUF
                if k >= NBUF:
                    wh[k - NBUF].wait()
                gh[k] = pltpu.async_copy(
                    table.at[idx_v.at[pl.ds(k * chunk, chunk)]], bufs[b], gsems[b]
                )
                if k >= LAG:
                    writeback(k - LAG)
            for j in range(max(0, nch - LAG), nch):
                writeback(j)
            for j in range(max(0, nch - NBUF), nch):
                wh[j].wait()

    return pl.kernel(
        body,
        mesh=mesh,
        out_type=[
            jax.ShapeDtypeStruct((B, H), jnp.int32),
            jax.ShapeDtypeStruct((B, H), jnp.int32),
        ],
        scratch_types=(
            [pltpu.VMEM((bpw,), jnp.int32)]
            + [pltpu.VMEM((chunk, H), jnp.int32)] * NBUF
            + [pltpu.SemaphoreType.DMA] * (2 * NBUF)
        ),
    )


def _sc_gather(table, idx, B):
    return _make_sc_gather(table.shape[0], B)(table, idx)


def _up_mm_body(x_ref, w_ref, b_ref, o_ref):
    z = (
        jnp.dot(x_ref[...].astype(jnp.bfloat16), w_ref[...],
                preferred_element_type=jnp.float32)
        + b_ref[...]
    )
    o_ref[...] = _pack(z[:, :H], z[:, H:])


def _assemble_body(ge_ref, go_ref, sl_ref, o_ref):
    i = pl.program_id(0)

    @pl.when(i < TOP_PAD // MBLK)
    def _top():
        o_ref[...] = ge_ref[...]

    @pl.when(i >= TOP_PAD // MBLK)
    def _down():
        ge = ge_ref[...]
        go = go_ref[...]
        e = jnp.concatenate([_unpack_lo(ge), _unpack_hi(ge)], axis=1)
        o = jnp.concatenate([_unpack_lo(go), _unpack_hi(go)], axis=1)
        left = jnp.dot(e, sl_ref[...], preferred_element_type=jnp.float32)
        right = jnp.dot(o, sl_ref[...], preferred_element_type=jnp.float32)
        o_ref[...] = _pack(left, right)


NPADH = NPAD // 2
B3H = B3 // 2
NBLKH = NPADH // MBLK


def _make_conv_mm_body(row0):
    def _conv_mm_body(g0, g1, g2, g3, g4, g5, g6, wl_ref, wh_ref, b_ref,
                      z_ref, st_ref, acc_ref):
        i = pl.program_id(0)
        grefs = (g0, g1, g2, g3, g4, g5, g6)
        z = b_ref[...]
        for k in range(7):
            g = grefs[k][...]
            wl = wl_ref[k * H:(k + 1) * H, :]
            wh = wh_ref[k * H:(k + 1) * H, :]
            z = z + jnp.dot(_unpack_lo(g), wl, preferred_element_type=jnp.float32)
            z = z + jnp.dot(_unpack_hi(g), wh, preferred_element_type=jnp.float32)
        z_ref[...] = _pack(z[:, :H], z[:, H:])
        rows = row0 + i * MBLK + lax.broadcasted_iota(jnp.int32, (MBLK, 1), 0)
        zm = jnp.where(rows < NEW, z, 0.0)

        @pl.when(i == 0)
        def _init():
            acc_ref[...] = jnp.zeros_like(acc_ref)

        acc_ref[0:1, :] += jnp.sum(zm, axis=0, keepdims=True)
        acc_ref[1:2, :] += jnp.sum(zm * zm, axis=0, keepdims=True)

        @pl.when(i == NBLKH - 1)
        def _fin():
            st_ref[...] = acc_ref[...]

    return _conv_mm_body


def _bn_act_body(za_ref, zb_ref, sta_ref, stb_ref, gam_ref, bet_ref, o_ref):
    i = pl.program_id(0)
    zp = jnp.where(i < NBLKH, za_ref[...], zb_ref[...])
    z = jnp.concatenate([_unpack_lo(zp), _unpack_hi(zp)], axis=1)
    st = sta_ref[...] + stb_ref[...]
    inv_n = 1.0 / NEW
    mean = st[0:1, :] * inv_n
    var = st[1:2, :] * inv_n - mean * mean
    scale = gam_ref[...] * lax.rsqrt(var + 1e-5)
    shift = bet_ref[...] - mean * scale
    a = z * scale + shift
    a = jnp.where(a >= 0, a, 0.2 * a)
    if o_ref.shape[1] == H:
        o_ref[...] = _pack(a[:, :H], a[:, H:])
    else:
        o_ref[...] = a


def _up_matmul(x1p, W_up, b_up):
    return pl.pallas_call(
        _up_mm_body,
        grid=(7,),
        in_specs=[
            pl.BlockSpec((M1, IN_CH), lambda j: (0, 0)),
            pl.BlockSpec((IN_CH, C), lambda j: (0, j)),
            pl.BlockSpec((1, C), lambda j: (0, j)),
        ],
        out_specs=pl.BlockSpec((M1, H), lambda j: (j, 0)),
        out_shape=jax.ShapeDtypeStruct((7 * M1, H), jnp.int32),
    )(x1p, W_up.astype(jnp.bfloat16), b_up.reshape(1, K7))


def _assemble_x(ge, go, sl):
    return pl.pallas_call(
        _assemble_body,
        grid=(NBLK,),
        in_specs=[
            pl.BlockSpec((MBLK, H), lambda i: (i, 0)),
            pl.BlockSpec((MBLK, H), lambda i: (i, 0)),
            pl.BlockSpec((C, H), lambda i: (0, 0)),
        ],
        out_specs=pl.BlockSpec((MBLK, H), lambda i: (i, 0)),
        out_shape=jax.ShapeDtypeStruct((NPAD, H), jnp.int32),
    )(ge, go, sl)


def _conv_matmul_half(g, W_lo, W_hi, b, row0):
    gspecs = [
        pl.BlockSpec((MBLK, H), (lambda i, kk=k: (kk * NBLKH + i, 0)))
        for k in range(7)
    ]
    return pl.pallas_call(
        _make_conv_mm_body(row0),
        grid=(NBLKH,),
        in_specs=gspecs + [
            pl.BlockSpec((KP, C), lambda i: (0, 0)),
            pl.BlockSpec((KP, C), lambda i: (0, 0)),
            pl.BlockSpec((1, C), lambda i: (0, 0)),
        ],
        out_specs=[
            pl.BlockSpec((MBLK, H), lambda i: (i, 0)),
            pl.BlockSpec((2, C), lambda i: (0, 0)),
        ],
        out_shape=[
            jax.ShapeDtypeStruct((NPADH, H), jnp.int32),
            jax.ShapeDtypeStruct((2, C), jnp.float32),
        ],
        scratch_shapes=[pltpu.VMEM((2, C), jnp.float32)],
    )(*([g] * 7), W_lo, W_hi, b.reshape(1, C))


def _conv(x_table, nidxa, nidxb, W, b):
    wl, wh = _split_w(W)
    ga = _sc_gather(x_table, nidxa, B3H)
    gb = _sc_gather(x_table, nidxb, B3H)
    za, sta = _conv_matmul_half(ga, wl, wh, b, 0)
    zb, stb = _conv_matmul_half(gb, wl, wh, b, NPADH)
    return za, zb, sta, stb


def _bn_act(za, zb, sta, stb, gamma, beta, packed, out_rows):
    nb = (out_rows + MBLK - 1) // MBLK
    return pl.pallas_call(
        _bn_act_body,
        grid=(nb,),
        in_specs=[
            pl.BlockSpec((MBLK, H), lambda i: (jnp.minimum(i, NBLKH - 1), 0)),
            pl.BlockSpec((MBLK, H), lambda i: (jnp.maximum(i - NBLKH, 0), 0)),
            pl.BlockSpec((2, C), lambda i: (0, 0)),
            pl.BlockSpec((2, C), lambda i: (0, 0)),
            pl.BlockSpec((1, C), lambda i: (0, 0)),
            pl.BlockSpec((1, C), lambda i: (0, 0)),
        ],
        out_specs=pl.BlockSpec((MBLK, H if packed else C), lambda i: (i, 0)),
        out_shape=jax.ShapeDtypeStruct(
            (out_rows, H if packed else C), jnp.int32 if packed else jnp.float32
        ),
    )(za, zb, sta, stb, gamma.reshape(1, C), beta.reshape(1, C))


def _split_w(W):
    w4 = W.reshape(7, 2, H, C)
    return w4[:, 0].reshape(KP, C), w4[:, 1].reshape(KP, C)


def kernel(x1, W_up, b_up, W_c1, b_c1, gamma1, beta1, W_c2, b_c2, gamma2,
           beta2, upconv_top_index, upconv_down_index, neigh_orders):
    i32 = jnp.int32
    top = upconv_top_index.astype(i32)
    dn = upconv_down_index.astype(i32).reshape(-1, 2)
    neigh = neigh_orders.astype(i32)

    def kmaj_up(r):
        return (r % 7) * M1 + r // 7

    zpad_top = jnp.arange(SHIFT, dtype=i32)
    zpad_dn = jnp.arange(NPAD - TOP_PAD - DOWN, dtype=i32)
    eidx = jnp.concatenate([kmaj_up(top), zpad_top, kmaj_up(dn[:, 0]), zpad_dn])
    oidx = jnp.concatenate([kmaj_up(top), zpad_top, kmaj_up(dn[:, 1]), zpad_dn])

    padrows = (jnp.arange((NPAD - NEW) * 7, dtype=i32) % NPAD).reshape(-1, 7)
    full = jnp.concatenate([neigh.reshape(NEW, 7), padrows], axis=0)
    ft = full.T
    ft1 = jnp.where(ft >= RAW, ft + SHIFT, ft)
    n1a = ft1[:, :NPADH].reshape(B3H)
    n1b = ft1[:, NPADH:].reshape(B3H)
    n2a = ft[:, :NPADH].reshape(B3H)
    n2b = ft[:, NPADH:].reshape(B3H)

    ccol = jnp.arange(C)[:, None] // 2
    krow = jnp.arange(H)[None, :]
    sl = jnp.where(ccol == krow, 0.5, 0.0).astype(jnp.float32)

    x1p = jnp.pad(x1, ((0, M1 - RAW), (0, 0)))

    up_flat = _up_matmul(x1p, W_up, b_up)

    ge, go = _make_sc_gather2(up_flat.shape[0], NPAD)(up_flat, eidx, oidx)
    x = _assemble_x(ge, go, sl)

    z1a, z1b, st1a, st1b = _conv(x, n1a, n1b, W_c1, b_c1)
    a1 = _bn_act(z1a, z1b, st1a, st1b, gamma1, beta1, True, NPAD)

    z2a, z2b, st2a, st2b = _conv(a1, n2a, n2b, W_c2, b_c2)
    return _bn_act(z2a, z2b, st2a, st2b, gamma2, beta2, False, NEW)
